# Initial kernel scaffold; baseline (speedup 1.0000x reference)
import jax
import jax.numpy as jnp
from jax.experimental import pallas as pl


def kernel(x, edge_index, W1, att_src1, att_dst1, b1, W2, att_src2, att_dst2, b2):
    raise NotImplementedError("write your pallas kernel here")


# speedup vs baseline: 25.3008x; 25.3008x over previous
import functools

import jax
import jax.numpy as jnp
from jax import lax
from jax.experimental import pallas as pl
from jax.experimental.pallas import tpu as pltpu
from jax.experimental.pallas import tpu_sc as plsc

_N = 10000
_E = 320000
_D_IN = 128
_H1, _C1 = 8, 8
_H2, _C2 = 1, 128
_EPS = 1e-16
_SLOPE = 0.2
_NT = 16


def _attn_table(a_s, a_d, hh):
    n = a_s.shape[0]
    pad = jnp.full((n, 16 - 2 * hh), -1e30, jnp.float32)
    return jnp.concatenate([a_s, a_d, pad], axis=1)


def _tc_pre_body(x_ref, w_ref, as_ref, ad_ref, h_out, at_out):
    h = jnp.dot(x_ref[...], w_ref[...], preferred_element_type=jnp.float32)
    a_s = jnp.dot(h, as_ref[...], preferred_element_type=jnp.float32)
    a_d = jnp.dot(h, ad_ref[...], preferred_element_type=jnp.float32)
    d = h.shape[1] // 2
    hh = a_s.shape[1] // 2
    h_out[...] = jnp.stack([h[:, :d], h[:, d:]])
    at_out[...] = jnp.stack(
        [_attn_table(a_s[:, :hh], a_d[:, :hh], hh),
         _attn_table(a_s[:, hh:], a_d[:, hh:], hh)])


def _row_block(n):
    for b in (2000, 2500, 1250, 1000, 500, 250, 200, 100):
        if n % b == 0:
            return b
    return n


def _tc_pre(x, w1, as_mat, ad_mat):
    n = x.shape[0]
    d = w1.shape[1]
    bn = _row_block(n)
    h = as_mat.shape[1]
    return pl.pallas_call(
        _tc_pre_body,
        grid=(n // bn,),
        in_specs=[
            pl.BlockSpec((bn, x.shape[1]), lambda i: (i, 0)),
            pl.BlockSpec((x.shape[1], d), lambda i: (0, 0)),
            pl.BlockSpec((d, h), lambda i: (0, 0)),
            pl.BlockSpec((d, h), lambda i: (0, 0)),
        ],
        out_specs=[
            pl.BlockSpec((2, bn, d // 2), lambda i: (0, i, 0)),
            pl.BlockSpec((2, bn, 16), lambda i: (0, i, 0)),
        ],
        out_shape=[
            jax.ShapeDtypeStruct((2, n, d // 2), jnp.float32),
            jax.ShapeDtypeStruct((2, n, 16), jnp.float32),
        ],
    )(x, w1, as_mat, ad_mat)


def _tc_mid_body(o1_ref, b1_ref, w2_ref, a2s_ref, a2d_ref,
                 h2_out, at2_out, *, dh, hh, cc):
    parts = []
    for c in (0, 1):
        o = o1_ref[c, :, :dh]
        den = o1_ref[c, :, dh:dh + hh]
        n = o.shape[0]
        o3 = o.reshape(n, hh, cc)
        den3 = den[:, :, None] + _EPS
        parts.append((o3 / den3).reshape(n, dh))
    hraw = jnp.concatenate(parts, axis=1) + b1_ref[...][None, :]
    h1 = jnp.where(hraw > 0, hraw, jnp.exp(jnp.minimum(hraw, 0.0)) - 1.0)
    h2 = jnp.dot(h1, w2_ref[...], preferred_element_type=jnp.float32)
    a2s = jnp.dot(h2, a2s_ref[...], preferred_element_type=jnp.float32)
    a2d = jnp.dot(h2, a2d_ref[...], preferred_element_type=jnp.float32)
    d2 = h2.shape[1] // 2
    h2_out[...] = jnp.stack([h2[:, :d2], h2[:, d2:]])
    at2 = _attn_table(a2s, a2d, 1)
    at2_out[...] = jnp.stack([at2, at2])


def _tc_mid(o1, b1, w2, a2s_mat, a2d_mat, dh, hh, cc):
    n = o1.shape[1]
    d2 = w2.shape[1]
    bn = _row_block(n)
    dhp = o1.shape[2]
    return pl.pallas_call(
        functools.partial(_tc_mid_body, dh=dh, hh=hh, cc=cc),
        grid=(n // bn,),
        in_specs=[
            pl.BlockSpec((2, bn, dhp), lambda i: (0, i, 0)),
            pl.BlockSpec((2 * dh,), lambda i: (0,)),
            pl.BlockSpec((2 * dh, d2), lambda i: (0, 0)),
            pl.BlockSpec((d2, 1), lambda i: (0, 0)),
            pl.BlockSpec((d2, 1), lambda i: (0, 0)),
        ],
        out_specs=[
            pl.BlockSpec((2, bn, d2 // 2), lambda i: (0, i, 0)),
            pl.BlockSpec((2, bn, 16), lambda i: (0, i, 0)),
        ],
        out_shape=[
            jax.ShapeDtypeStruct((2, n, d2 // 2), jnp.float32),
            jax.ShapeDtypeStruct((2, n, 16), jnp.float32),
        ],
    )(o1, b1, w2, a2s_mat, a2d_mat)


def _tc_fin_body(o2_ref, b2_ref, out_ref, *, dh):
    parts = []
    for c in (0, 1):
        o = o2_ref[c, :, :dh]
        den = o2_ref[c, :, dh:dh + 1] + _EPS
        parts.append(o / den)
    out_ref[...] = jnp.concatenate(parts, axis=1) + b2_ref[...][None, :]


def _tc_fin(o2, b2, dh):
    n = o2.shape[1]
    bn = _row_block(n)
    dhp = o2.shape[2]
    return pl.pallas_call(
        functools.partial(_tc_fin_body, dh=dh),
        grid=(n // bn,),
        in_specs=[
            pl.BlockSpec((2, bn, dhp), lambda i: (0, i, 0)),
            pl.BlockSpec((2 * dh,), lambda i: (0,)),
        ],
        out_specs=pl.BlockSpec((bn, 2 * dh), lambda i: (i, 0)),
        out_shape=jax.ShapeDtypeStruct((n, 2 * dh), jnp.float32),
    )(o2, b2)


_GDN = lax.GatherDimensionNumbers(
    offset_dims=(), collapsed_slice_dims=(0,), start_index_map=(0,))


def _take16(v, idx16):
    return lax.gather(v, idx16[:, None], _GDN, (1,),
                      mode=lax.GatherScatterMode.PROMISE_IN_BOUNDS)

def _make_sc_edge(n, e_real, e_pad, dh, hh, cc, k):
    dhp = dh + 16
    per_tile = e_pad // _NT
    nchunks = per_tile // k
    rpt = n // _NT // 8 * 8
    tail = n - _NT * rpt
    zr = 24
    mesh = plsc.VectorSubcoreMesh(core_axis_name="c", subcore_axis_name="s",
                                  num_cores=2, num_subcores=_NT)

    @functools.partial(
        pl.kernel,
        out_type=jax.ShapeDtypeStruct((2, n, dhp), jnp.float32),
        mesh=mesh,
        compiler_params=pltpu.CompilerParams(needs_layout_passes=False,
                                             use_tc_tiling_on_sc=False),
        scratch_types=[
            pltpu.VMEM_SHARED((n, dh), jnp.float32),
            pltpu.VMEM_SHARED((n, dhp), jnp.float32),
            pltpu.VMEM_SHARED((n, 16), jnp.float32),
            pltpu.VMEM((k,), jnp.int32),
            pltpu.VMEM((k,), jnp.int32),
            pltpu.VMEM((k, 16), jnp.float32),
            pltpu.VMEM((k, 16), jnp.float32),
            pltpu.VMEM((k, dh), jnp.float32),
            pltpu.VMEM((k, dhp), jnp.float32),
            pltpu.VMEM((zr, dhp), jnp.float32),
        ],
    )
    def edge_kernel(h_hbm, at_hbm, src_hbm, dst_hbm, out_hbm,
                    h_s, out_s, a_s, src_i, dst_i, asr, adr, rows, msg,
                    zb):
        c = lax.axis_index("c")
        s = lax.axis_index("s")
        iota16 = lax.broadcasted_iota(jnp.int32, (16,), 0)
        zv = jnp.zeros((16,), jnp.float32)
        shift_idx = jnp.minimum(iota16 + hh, 15)
        seg_idx = [(iota16 + j2 * 16) // cc for j2 in range(dh // 16)]

        for r in range(zr):
            for j in range(dhp // 16):
                zb[r, pl.ds(j * 16, 16)] = zv

        row0 = s * rpt
        nz, rem = divmod(rpt, zr)
        for z in range(nz):
            pltpu.sync_copy(zb, out_s.at[pl.ds(row0 + z * zr, zr)])
        if rem:
            pltpu.sync_copy(zb.at[pl.ds(0, rem)],
                            out_s.at[pl.ds(row0 + nz * zr, rem)])
        pltpu.sync_copy(h_hbm.at[c, pl.ds(row0, rpt)],
                        h_s.at[pl.ds(row0, rpt)])
        pltpu.sync_copy(at_hbm.at[c, pl.ds(row0, rpt)],
                        a_s.at[pl.ds(row0, rpt)])
        if tail:
            @pl.when(s == _NT - 1)
            def _tail_init():
                t0 = _NT * rpt
                pltpu.sync_copy(zb.at[pl.ds(0, tail)],
                                out_s.at[pl.ds(t0, tail)])
                pltpu.sync_copy(h_hbm.at[c, pl.ds(t0, tail)],
                                h_s.at[pl.ds(t0, tail)])
                pltpu.sync_copy(at_hbm.at[c, pl.ds(t0, tail)],
                                a_s.at[pl.ds(t0, tail)])
        plsc.subcore_barrier()

        tile_base = s * per_tile

        def chunk_body(g, carry):
            base = tile_base + g * k
            pltpu.sync_copy(src_hbm.at[pl.ds(base, k)], src_i)
            pltpu.sync_copy(dst_hbm.at[pl.ds(base, k)], dst_i)
            pltpu.sync_copy(a_s.at[src_i], asr)
            pltpu.sync_copy(a_s.at[dst_i], adr)
            pltpu.sync_copy(h_s.at[src_i], rows)

            def edge_body(ei, gc):
                sv = asr[ei, pl.ds(0, 16)]
                dv = adr[ei, pl.ds(0, 16)]
                al = sv + _take16(dv, shift_idx)
                al = jnp.where(al > 0, al, _SLOPE * al)
                ex = jnp.exp(al)
                if e_pad != e_real:
                    ex = ex * (base + ei < e_real).astype(jnp.float32)
                msg[ei, pl.ds(dh, 16)] = ex
                for j2 in range(dh // 16):
                    sc16 = _take16(ex, seg_idx[j2])
                    msg[ei, pl.ds(j2 * 16, 16)] = (
                        rows[ei, pl.ds(j2 * 16, 16)] * sc16)
                return gc
            lax.fori_loop(0, k, edge_body, 0)
            pltpu.sync_copy(msg, out_s.at[dst_i], add=True)
            return carry
        lax.fori_loop(0, nchunks, chunk_body, 0)
        plsc.subcore_barrier()

        pltpu.sync_copy(out_s.at[pl.ds(row0, rpt)],
                        out_hbm.at[c, pl.ds(row0, rpt)])
        if tail:
            @pl.when(s == _NT - 1)
            def _tail_out():
                t0 = _NT * rpt
                pltpu.sync_copy(out_s.at[pl.ds(t0, tail)],
                                out_hbm.at[c, pl.ds(t0, tail)])

    return edge_kernel


def _pick_k(e_pad, pref):
    per_tile = e_pad // _NT
    for k in (pref, 160, 80, 48, 16):
        if k % 16 == 0 and per_tile % k == 0:
            return k
    return 16


def kernel(x, edge_index, W1, att_src1, att_dst1, b1, W2, att_src2,
           att_dst2, b2):
    n = x.shape[0]
    e = edge_index.shape[1]
    h1, c1 = att_src1.shape[1], att_src1.shape[2]
    h2, c2 = att_src2.shape[1], att_src2.shape[2]
    d1, d2 = h1 * c1, h2 * c2

    src = edge_index[0].astype(jnp.int32)
    dst = edge_index[1].astype(jnp.int32)
    e_pad = e
    quant = _NT * 16
    if e_pad % quant:
        e_pad = (e // quant + 1) * quant
    if e_pad != e:
        src = jnp.concatenate([src, jnp.zeros((e_pad - e,), jnp.int32)])
        dst = jnp.concatenate([dst, jnp.zeros((e_pad - e,), jnp.int32)])

    eye1 = jnp.eye(h1, dtype=jnp.float32)
    as1 = (att_src1[0][:, :, None] * eye1[:, None, :]).reshape(d1, h1)
    ad1 = (att_dst1[0][:, :, None] * eye1[:, None, :]).reshape(d1, h1)
    a2s_mat = att_src2.reshape(d2, h2)
    a2d_mat = att_dst2.reshape(d2, h2)

    h1s, at1 = _tc_pre(x, W1, as1, ad1)

    k1 = _pick_k(e_pad, 160)
    sc1 = _make_sc_edge(n, e, e_pad, d1 // 2, h1 // 2, c1, k1)
    o1 = sc1(h1s, at1, src, dst)

    h2s, at2 = _tc_mid(o1, b1, W2, a2s_mat, a2d_mat,
                       d1 // 2, h1 // 2, c1)

    k2 = _pick_k(e_pad, 80)
    sc2 = _make_sc_edge(n, e, e_pad, d2 // 2, max(h2 // 2, 1),
                        d2 // 2 // max(h2 // 2, 1), k2)
    o2 = sc2(h2s, at2, src, dst)

    return _tc_fin(o2, b2, d2 // 2)
